# baseline (speedup 1.0000x reference)
import functools

import jax
import jax.numpy as jnp
from jax import lax
from jax.experimental import pallas as pl
from jax.experimental.pallas import tpu as pltpu
from jax.experimental.pallas import tpu_sc as plsc

_B = 128
_S = 512
_D = 768
_NW = 32
_SBLK = _S // _NW
_CB = 4
_C = _CB * _SBLK
_NCHUNK = _B // _CB
_LANES = 16
_KD = _D // _LANES


_DNUMS = lax.GatherDimensionNumbers(
    offset_dims=(), collapsed_slice_dims=(0,), start_index_map=(0,))


def _permute(v, idx):
    return lax.gather(v, idx.reshape(_LANES, 1), _DNUMS, (1,),
                      mode=lax.GatherScatterMode.PROMISE_IN_BOUNDS)


def _allsum(v):
    lanes = lax.iota(jnp.int32, _LANES)
    for sh in (8, 4, 2, 1):
        v = v + _permute(v, lanes ^ sh)
    return v


def _rsqrt(x):
    i = lax.bitcast_convert_type(x, jnp.int32)
    i = 0x5F3759DF - lax.shift_right_arithmetic(i, 1)
    y = lax.bitcast_convert_type(i, jnp.float32)
    for _ in range(3):
        y = y * (1.5 - 0.5 * x * y * y)
    return y


def _body(x_hbm, seg_hbm, tok_hbm, segtab_hbm, pos_hbm, gamma_hbm, beta_hbm,
          out_hbm, idx_v, seg_v, rows_v, cache_v, delta_v, segtab_v, gamma_v,
          beta_v, sem):
    wid = lax.axis_index("s") * 2 + lax.axis_index("c")
    s0 = wid * _SBLK

    pltpu.sync_copy(gamma_hbm, gamma_v)
    pltpu.sync_copy(beta_hbm, beta_v)
    pltpu.sync_copy(segtab_hbm, segtab_v)
    pltpu.sync_copy(pos_hbm.at[pl.ds(s0, _SBLK)], cache_v)

    for k in range(_KD):
        dsl = pl.ds(k * _LANES, _LANES)
        delta_v[dsl] = segtab_v[1, dsl] - segtab_v[0, dsl]

    def add_seg(jj, carry):
        for k in range(_KD):
            dsl = pl.ds(k * _LANES, _LANES)
            cache_v[jj, dsl] = cache_v[jj, dsl] + segtab_v[0, dsl]
        return carry

    lax.fori_loop(0, _SBLK, add_seg, 0)

    def chunk(g, carry):
        b0 = g * _CB
        for u in range(_CB):
            off = (b0 + u) * _S + s0
            pltpu.sync_copy(x_hbm.at[pl.ds(off, _SBLK)],
                            idx_v.at[pl.ds(u * _SBLK, _SBLK)])
            pltpu.sync_copy(seg_hbm.at[pl.ds(off, _SBLK)],
                            seg_v.at[pl.ds(u * _SBLK, _SBLK)])
        pltpu.async_copy(tok_hbm.at[idx_v], rows_v, sem).wait()

        def token(t, tc):
            jj = lax.rem(t, _SBLK)
            sve = seg_v[pl.ds(t - jj, _LANES)]
            sv = _permute(sve, jnp.broadcast_to(jj, (_LANES,)))
            segf = sv.astype(jnp.float32)
            acc = jnp.zeros((_LANES,), jnp.float32)
            acc2 = jnp.zeros((_LANES,), jnp.float32)
            for k in range(_KD):
                dsl = pl.ds(k * _LANES, _LANES)
                v = (rows_v[t, dsl] + cache_v[jj, dsl]
                     + segf * delta_v[dsl])
                rows_v[t, dsl] = v
                acc = acc + v
                acc2 = acc2 + v * v
            mean = _allsum(acc) * (1.0 / _D)
            m2 = _allsum(acc2) * (1.0 / _D)
            inv = _rsqrt(m2 - mean * mean + 1e-5)
            for k in range(_KD):
                dsl = pl.ds(k * _LANES, _LANES)
                v = rows_v[t, dsl]
                rows_v[t, dsl] = ((v - mean) * inv * gamma_v[dsl]
                                  + beta_v[dsl])
            return tc

        lax.fori_loop(0, _C, token, 0)
        for u in range(_CB):
            off = (b0 + u) * _S + s0
            pltpu.sync_copy(rows_v.at[pl.ds(u * _SBLK, _SBLK)],
                            out_hbm.at[pl.ds(off, _SBLK)])
        return carry

    lax.fori_loop(0, _NCHUNK, chunk, 0)


@jax.jit
def _run(xf, sf, tok_table, seg_table, pos_table, gamma, beta):
    call = functools.partial(
        pl.kernel,
        out_type=jax.ShapeDtypeStruct((_B * _S, _D), jnp.float32),
        mesh=plsc.VectorSubcoreMesh(core_axis_name="c", subcore_axis_name="s"),
        scratch_types=[
            pltpu.VMEM((_C,), jnp.int32),
            pltpu.VMEM((_C,), jnp.int32),
            pltpu.VMEM((_C, _D), jnp.float32),
            pltpu.VMEM((_SBLK, _D), jnp.float32),
            pltpu.VMEM((_D,), jnp.float32),
            pltpu.VMEM((2, _D), jnp.float32),
            pltpu.VMEM((_D,), jnp.float32),
            pltpu.VMEM((_D,), jnp.float32),
            pltpu.SemaphoreType.DMA,
        ],
    )(_body)
    return call(xf, sf, tok_table, seg_table, pos_table, gamma, beta)


def kernel(x, seg, tok_table, seg_table, pos_table, gamma, beta):
    xf = x.reshape(-1)
    sf = seg.reshape(-1)
    out = _run(xf, sf, tok_table, seg_table, pos_table, gamma, beta)
    return out.reshape(x.shape[0], x.shape[1], tok_table.shape[1])
